# baseline (speedup 1.0000x reference)
import functools

import jax
import jax.numpy as jnp
from jax import lax
from jax.experimental import pallas as pl
from jax.experimental.pallas import tpu as pltpu
from jax.experimental.pallas import tpu_sc as plsc

NUM_EMBS = 1000000
EMB_SIZE = 32
BATCH = 16384
HIST = 50

_NC = 2
_NS = 16
_NW = _NC * _NS

_SPW = BATCH // _NW
_G = 16
_NB = 2
_NGRP = _SPW // _G
_IW = 128

_PREP_BLK = 2048

_TBLK = 6400
_TROWS = _TBLK * EMB_SIZE // 128


def _tprep_body(tab_ref, out_ref):
  y = tab_ref[...].T.reshape(_TROWS, 4, EMB_SIZE)
  eye = jnp.eye(128, dtype=jnp.float32)
  acc = jnp.zeros((_TROWS, 128), jnp.float32)
  for a in range(4):
    acc = acc + jax.lax.dot(y[:, a, :],
                            eye[EMB_SIZE * a:EMB_SIZE * (a + 1), :],
                            preferred_element_type=jnp.float32)
  out_ref[...] = acc


_tprep = pl.pallas_call(
    _tprep_body,
    grid=(pl.cdiv(NUM_EMBS, _TBLK),),
    in_specs=[pl.BlockSpec((EMB_SIZE, _TBLK), lambda i: (0, i))],
    out_specs=pl.BlockSpec((_TROWS, 128), lambda i: (i, 0)),
    out_shape=jax.ShapeDtypeStruct((NUM_EMBS * EMB_SIZE // 128, 128),
                                   jnp.float32),
)


def _prep_body(idx_ref, out_ref):
  x = idx_ref[...]
  pad = jnp.broadcast_to(x[:, :1], (_PREP_BLK, _IW - HIST))
  out_ref[...] = jnp.concatenate([x, pad], axis=1)


_prep = pl.pallas_call(
    _prep_body,
    grid=(BATCH // _PREP_BLK,),
    in_specs=[pl.BlockSpec((_PREP_BLK, HIST), lambda i: (i, 0))],
    out_specs=pl.BlockSpec((_PREP_BLK, _IW), lambda i: (i, 0)),
    out_shape=jax.ShapeDtypeStruct((BATCH, _IW), jnp.int32),
)


def _make_gather():
  mesh = plsc.VectorSubcoreMesh(core_axis_name="c", subcore_axis_name="s")

  @functools.partial(
      pl.kernel,
      mesh=mesh,
      compiler_params=pltpu.CompilerParams(use_tc_tiling_on_sc=False),
      out_type=jax.ShapeDtypeStruct((BATCH, HIST, EMB_SIZE), jnp.float32),
      scratch_types=[
          pltpu.VMEM((_SPW, 56), jnp.int32),
          pltpu.VMEM((_NB, _G, 56, EMB_SIZE), jnp.float32),
          pltpu.SemaphoreType.DMA,
          pltpu.SemaphoreType.DMA((_NB,)),
      ],
  )
  def gather_kernel(idx_hbm, tab_hbm, out_hbm, idx_v, rows_v, gsem, wsem):
    wid = lax.axis_index("s") * _NC + lax.axis_index("c")
    base_samp = wid * _SPW
    pltpu.sync_copy(
        idx_hbm.at[pl.ds(base_samp, _SPW), pl.ds(0, 56)], idx_v)

    def group_pair(gp, carry):
      for b in range(_NB):
        g = gp * _NB + b
        buf = rows_v.at[b]
        @pl.when(gp > 0)
        def _():
          pltpu.make_async_copy(
              buf.at[:, pl.ds(0, HIST)],
              out_hbm.at[pl.ds(base_samp, _G)], wsem.at[b]).wait()

        handles = []
        for j in range(_G):
          handles.append(pltpu.async_copy(
              tab_hbm.at[idx_v.at[g * _G + j]], buf.at[j], gsem))
        for h in handles:
          h.wait()
        pltpu.async_copy(buf.at[:, pl.ds(0, HIST)],
                         out_hbm.at[pl.ds(base_samp + g * _G, _G)],
                         wsem.at[b])
      return carry

    lax.fori_loop(0, _NGRP // _NB, group_pair, 0)
    for b in range(_NB):
      pltpu.make_async_copy(
          rows_v.at[b, :, pl.ds(0, HIST)],
          out_hbm.at[pl.ds(base_samp, _G)], wsem.at[b]).wait()

  return gather_kernel


_gather = _make_gather()


def kernel(indices, table):
  idx_padded = _prep(indices.astype(jnp.int32))
  table_flat = _tprep(table.T).reshape(NUM_EMBS, EMB_SIZE)
  return _gather(idx_padded, table_flat)


# speedup vs baseline: 1.0067x; 1.0067x over previous
import functools

import jax
import jax.numpy as jnp
from jax import lax
from jax.experimental import pallas as pl
from jax.experimental.pallas import tpu as pltpu
from jax.experimental.pallas import tpu_sc as plsc

NUM_EMBS = 1000000
EMB_SIZE = 32
BATCH = 16384
HIST = 50

_NC = 2
_NS = 16
_NW = _NC * _NS

_SPW = BATCH // _NW
_G = 16
_NB = 2
_NGRP = _SPW // _G
_IW = 128

_PREP_BLK = 2048

_TBLK = 12800
_TROWS = _TBLK * EMB_SIZE // 128


def _tprep_body(tab_ref, out_ref):
  y = tab_ref[...].T.reshape(_TROWS, 4, EMB_SIZE)
  eye = jnp.eye(128, dtype=jnp.float32)
  acc = jnp.zeros((_TROWS, 128), jnp.float32)
  for a in range(4):
    acc = acc + jax.lax.dot(y[:, a, :],
                            eye[EMB_SIZE * a:EMB_SIZE * (a + 1), :],
                            preferred_element_type=jnp.float32)
  out_ref[...] = acc


_tprep = pl.pallas_call(
    _tprep_body,
    grid=(pl.cdiv(NUM_EMBS, _TBLK),),
    in_specs=[pl.BlockSpec((EMB_SIZE, _TBLK), lambda i: (0, i))],
    out_specs=pl.BlockSpec((_TROWS, 128), lambda i: (i, 0)),
    out_shape=jax.ShapeDtypeStruct((NUM_EMBS * EMB_SIZE // 128, 128),
                                   jnp.float32),
)


def _prep_body(idx_ref, out_ref):
  x = idx_ref[...]
  pad = jnp.broadcast_to(x[:, :1], (_PREP_BLK, _IW - HIST))
  out_ref[...] = jnp.concatenate([x, pad], axis=1)


_prep = pl.pallas_call(
    _prep_body,
    grid=(BATCH // _PREP_BLK,),
    in_specs=[pl.BlockSpec((_PREP_BLK, HIST), lambda i: (i, 0))],
    out_specs=pl.BlockSpec((_PREP_BLK, _IW), lambda i: (i, 0)),
    out_shape=jax.ShapeDtypeStruct((BATCH, _IW), jnp.int32),
)


def _make_gather():
  mesh = plsc.VectorSubcoreMesh(core_axis_name="c", subcore_axis_name="s")

  @functools.partial(
      pl.kernel,
      mesh=mesh,
      compiler_params=pltpu.CompilerParams(use_tc_tiling_on_sc=False),
      out_type=jax.ShapeDtypeStruct((BATCH, HIST, EMB_SIZE), jnp.float32),
      scratch_types=[
          pltpu.VMEM((_SPW, 56), jnp.int32),
          pltpu.VMEM((_NB, _G, 56, EMB_SIZE), jnp.float32),
          pltpu.SemaphoreType.DMA,
          pltpu.SemaphoreType.DMA((_NB,)),
      ],
  )
  def gather_kernel(idx_hbm, tab_hbm, out_hbm, idx_v, rows_v, gsem, wsem):
    wid = lax.axis_index("s") * _NC + lax.axis_index("c")
    base_samp = wid * _SPW
    pltpu.sync_copy(
        idx_hbm.at[pl.ds(base_samp, _SPW), pl.ds(0, 56)], idx_v)

    def group_pair(gp, carry):
      for b in range(_NB):
        g = gp * _NB + b
        buf = rows_v.at[b]
        @pl.when(gp > 0)
        def _():
          pltpu.make_async_copy(
              buf.at[:, pl.ds(0, HIST)],
              out_hbm.at[pl.ds(base_samp, _G)], wsem.at[b]).wait()

        handles = []
        for j in range(_G):
          handles.append(pltpu.async_copy(
              tab_hbm.at[idx_v.at[g * _G + j]], buf.at[j], gsem))
        for h in handles:
          h.wait()
        pltpu.async_copy(buf.at[:, pl.ds(0, HIST)],
                         out_hbm.at[pl.ds(base_samp + g * _G, _G)],
                         wsem.at[b])
      return carry

    lax.fori_loop(0, _NGRP // _NB, group_pair, 0)
    for b in range(_NB):
      pltpu.make_async_copy(
          rows_v.at[b, :, pl.ds(0, HIST)],
          out_hbm.at[pl.ds(base_samp, _G)], wsem.at[b]).wait()

  return gather_kernel


_gather = _make_gather()


def kernel(indices, table):
  idx_padded = _prep(indices.astype(jnp.int32))
  table_flat = _tprep(table.T).reshape(NUM_EMBS, EMB_SIZE)
  return _gather(idx_padded, table_flat)
